# Initial kernel scaffold; baseline (speedup 1.0000x reference)
import jax
import jax.numpy as jnp
from jax.experimental import pallas as pl


def kernel(basic_block, edge_index, h0, c0, gcn_W, gcn_b, W_ih, W_hh, b_ih, b_hh):
    raise NotImplementedError("write your pallas kernel here")


# speedup vs baseline: 8.2716x; 8.2716x over previous
import functools

import jax
import jax.numpy as jnp
from jax import lax
from jax.experimental import pallas as pl
from jax.experimental.pallas import tpu as pltpu


H = 128


def _y_kernel(x_ref, w_ref, deg_ref, y_ref, dinv_ref):
    deg = deg_ref[0, 0, :] + 1.0
    dinv = lax.rsqrt(deg)
    xw = jnp.dot(x_ref[...], w_ref[...], preferred_element_type=jnp.float32)
    y_ref[...] = xw * dinv[:, None]
    dinv_ref[0, 0, :] = dinv


def _compute_y(x, gcn_W, deg):
    n = x.shape[0]
    blk = 1000
    grid = n // blk
    y, dinv = pl.pallas_call(
        _y_kernel,
        grid=(grid,),
        in_specs=[
            pl.BlockSpec((blk, H), lambda i: (i, 0)),
            pl.BlockSpec((H, H), lambda i: (0, 0)),
            pl.BlockSpec((1, 1, blk), lambda i: (i, 0, 0)),
        ],
        out_specs=[
            pl.BlockSpec((blk, H), lambda i: (i, 0)),
            pl.BlockSpec((1, 1, blk), lambda i: (i, 0, 0)),
        ],
        out_shape=[
            jax.ShapeDtypeStruct((n, H), jnp.float32),
            jax.ShapeDtypeStruct((grid, 1, blk), jnp.float32),
        ],
    )(x, gcn_W, deg.reshape(grid, 1, blk))
    return y, dinv


def _g_kernel(acc_ref, y_ref, dinv_ref, gb_ref, wih_ref, b_ref, g_ref):
    dinv = dinv_ref[0, 0, :]
    x = dinv[:, None] * (acc_ref[...] + y_ref[...]) + gb_ref[...]
    g_ref[...] = (
        lax.dot_general(x, wih_ref[...], (((1,), (1,)), ((), ())),
                        preferred_element_type=jnp.float32)
        + b_ref[...]
    )


def _compute_g(acc, y, dinv, gcn_b, W_ih, b_ih, b_hh):
    n = y.shape[0]
    blk = 1000
    grid = n // blk
    return pl.pallas_call(
        _g_kernel,
        grid=(grid,),
        in_specs=[
            pl.BlockSpec((blk, H), lambda i: (i, 0)),
            pl.BlockSpec((blk, H), lambda i: (i, 0)),
            pl.BlockSpec((1, 1, blk), lambda i: (i, 0, 0)),
            pl.BlockSpec((1, H), lambda i: (0, 0)),
            pl.BlockSpec((4 * H, H), lambda i: (0, 0)),
            pl.BlockSpec((1, 4 * H), lambda i: (0, 0)),
        ],
        out_specs=pl.BlockSpec((blk, 4 * H), lambda i: (i, 0)),
        out_shape=jax.ShapeDtypeStruct((n, 4 * H), jnp.float32),
    )(acc, y, dinv, gcn_b.reshape(1, H), W_ih,
      (b_ih + b_hh).reshape(1, 4 * H))


def _lstm_kernel(g_ref, whh_ref, h0_ref, c0_ref,
                 ys_ref, hn_ref, cn_ref, h_s, c_s, *, chunk):
    t = pl.program_id(0)

    @pl.when(t == 0)
    def _():
        h_s[...] = h0_ref[...]
        c_s[...] = c0_ref[...]

    whh = whh_ref[...]

    def step(i, carry):
        h, c = carry
        g = g_ref[pl.ds(i, 1), :] + lax.dot_general(
            h, whh, (((1,), (1,)), ((), ())),
            preferred_element_type=jnp.float32)
        ii = jax.nn.sigmoid(g[:, :H])
        ff = jax.nn.sigmoid(g[:, H:2 * H])
        gg = jnp.tanh(g[:, 2 * H:3 * H])
        oo = jax.nn.sigmoid(g[:, 3 * H:])
        c2 = ff * c + ii * gg
        h2 = oo * jnp.tanh(c2)
        ys_ref[pl.ds(i, 1), :] = h2
        return h2, c2

    h, c = lax.fori_loop(0, chunk, step, (h_s[...], c_s[...]))
    h_s[...] = h
    c_s[...] = c
    hn_ref[...] = h
    cn_ref[...] = c


def _run_lstm(G, W_hh, h0, c0):
    n = G.shape[0]
    chunk = 1000
    grid = n // chunk
    ys, hn, cn = pl.pallas_call(
        functools.partial(_lstm_kernel, chunk=chunk),
        grid=(grid,),
        in_specs=[
            pl.BlockSpec((chunk, 4 * H), lambda i: (i, 0)),
            pl.BlockSpec((4 * H, H), lambda i: (0, 0)),
            pl.BlockSpec((1, H), lambda i: (0, 0)),
            pl.BlockSpec((1, H), lambda i: (0, 0)),
        ],
        out_specs=[
            pl.BlockSpec((chunk, H), lambda i: (i, 0)),
            pl.BlockSpec((1, H), lambda i: (0, 0)),
            pl.BlockSpec((1, H), lambda i: (0, 0)),
        ],
        out_shape=[
            jax.ShapeDtypeStruct((n, H), jnp.float32),
            jax.ShapeDtypeStruct((1, H), jnp.float32),
            jax.ShapeDtypeStruct((1, H), jnp.float32),
        ],
        scratch_shapes=[
            pltpu.VMEM((1, H), jnp.float32),
            pltpu.VMEM((1, H), jnp.float32),
        ],
    )(G, W_hh, h0, c0)
    return ys, hn, cn


def _edge_pass(y, src, dst, n):
    deg_like = jax.ops.segment_sum(y[src], dst, num_segments=n)
    return deg_like


def kernel(basic_block, edge_index, h0, c0, gcn_W, gcn_b,
           W_ih, W_hh, b_ih, b_hh):
    n = basic_block.shape[0]
    src = edge_index[0]
    dst = edge_index[1]
    deg = jax.ops.segment_sum(
        jnp.ones_like(dst, dtype=jnp.float32), dst, num_segments=n)
    y, dinv = _compute_y(basic_block, gcn_W, deg)
    acc = _edge_pass(y, src, dst, n)
    G = _compute_g(acc, y, dinv, gcn_b, W_ih, b_ih, b_hh)
    ys, hn, cn = _run_lstm(G, W_hh, h0, c0)
    return ys, hn, cn


# speedup vs baseline: 9.1707x; 1.1087x over previous
import functools

import jax
import jax.numpy as jnp
from jax import lax
from jax.experimental import pallas as pl
from jax.experimental.pallas import tpu as pltpu


H = 128


def _y_kernel(x_ref, w_ref, deg_ref, y_ref, dinv_ref):
    deg = deg_ref[0, 0, :] + 1.0
    dinv = lax.rsqrt(deg)
    xw = jnp.dot(x_ref[...], w_ref[...], preferred_element_type=jnp.float32)
    y_ref[...] = xw * dinv[:, None]
    dinv_ref[0, 0, :] = dinv


def _compute_y(x, gcn_W, deg):
    n = x.shape[0]
    blk = 1000
    grid = n // blk
    y, dinv = pl.pallas_call(
        _y_kernel,
        grid=(grid,),
        in_specs=[
            pl.BlockSpec((blk, H), lambda i: (i, 0)),
            pl.BlockSpec((H, H), lambda i: (0, 0)),
            pl.BlockSpec((1, 1, blk), lambda i: (i, 0, 0)),
        ],
        out_specs=[
            pl.BlockSpec((blk, H), lambda i: (i, 0)),
            pl.BlockSpec((1, 1, blk), lambda i: (i, 0, 0)),
        ],
        out_shape=[
            jax.ShapeDtypeStruct((n, H), jnp.float32),
            jax.ShapeDtypeStruct((grid, 1, blk), jnp.float32),
        ],
    )(x, gcn_W, deg.reshape(grid, 1, blk))
    return y, dinv


def _g_kernel(acc_ref, y_ref, dinv_ref, gb_ref, wih_ref, b_ref, g_ref):
    dinv = dinv_ref[0, 0, :]
    x = dinv[:, None] * (acc_ref[...] + y_ref[...]) + gb_ref[...]
    g_ref[...] = (
        lax.dot_general(x, wih_ref[...], (((1,), (1,)), ((), ())),
                        preferred_element_type=jnp.float32)
        + b_ref[...]
    )


def _compute_g(acc, y, dinv, gcn_b, W_ih, b_ih, b_hh):
    n = y.shape[0]
    blk = 1000
    grid = n // blk
    return pl.pallas_call(
        _g_kernel,
        grid=(grid,),
        in_specs=[
            pl.BlockSpec((blk, H), lambda i: (i, 0)),
            pl.BlockSpec((blk, H), lambda i: (i, 0)),
            pl.BlockSpec((1, 1, blk), lambda i: (i, 0, 0)),
            pl.BlockSpec((1, H), lambda i: (0, 0)),
            pl.BlockSpec((4 * H, H), lambda i: (0, 0)),
            pl.BlockSpec((1, 4 * H), lambda i: (0, 0)),
        ],
        out_specs=pl.BlockSpec((blk, 4 * H), lambda i: (i, 0)),
        out_shape=jax.ShapeDtypeStruct((n, 4 * H), jnp.float32),
    )(acc, y, dinv, gcn_b.reshape(1, H), W_ih,
      (b_ih + b_hh).reshape(1, 4 * H))


def _lstm_kernel(g_ref, whht_ref, h0_ref, c0_ref,
                 ys_ref, hn_ref, cn_ref, h_s, c_s, *, nblk):
    t = pl.program_id(0)

    @pl.when(t == 0)
    def _():
        h_s[...] = h0_ref[...]
        c_s[...] = c0_ref[...]

    whht = whht_ref[...]

    def blk_step(j, carry):
        h, c = carry
        gblk = g_ref[j]
        rows = []
        for k in range(8):
            g = gblk[k:k + 1, :] + jnp.dot(
                h, whht, preferred_element_type=jnp.float32)
            ii = jax.nn.sigmoid(g[:, :H])
            ff = jax.nn.sigmoid(g[:, H:2 * H])
            gg = jnp.tanh(g[:, 2 * H:3 * H])
            oo = jax.nn.sigmoid(g[:, 3 * H:])
            c = ff * c + ii * gg
            h = oo * jnp.tanh(c)
            rows.append(h)
        ys_ref[j] = jnp.concatenate(rows, axis=0)
        return h, c

    h, c = lax.fori_loop(0, nblk, blk_step, (h_s[...], c_s[...]))
    h_s[...] = h
    c_s[...] = c
    hn_ref[...] = h
    cn_ref[...] = c


def _run_lstm(G, W_hh_T, h0, c0):
    n = G.shape[0]
    chunk = 1000
    grid = n // chunk
    nblk = chunk // 8
    G8 = G.reshape(n // 8, 8, 4 * H)
    ys, hn, cn = pl.pallas_call(
        functools.partial(_lstm_kernel, nblk=nblk),
        grid=(grid,),
        in_specs=[
            pl.BlockSpec((nblk, 8, 4 * H), lambda i: (i, 0, 0)),
            pl.BlockSpec((H, 4 * H), lambda i: (0, 0)),
            pl.BlockSpec((1, H), lambda i: (0, 0)),
            pl.BlockSpec((1, H), lambda i: (0, 0)),
        ],
        out_specs=[
            pl.BlockSpec((nblk, 8, H), lambda i: (i, 0, 0)),
            pl.BlockSpec((1, H), lambda i: (0, 0)),
            pl.BlockSpec((1, H), lambda i: (0, 0)),
        ],
        out_shape=[
            jax.ShapeDtypeStruct((n // 8, 8, H), jnp.float32),
            jax.ShapeDtypeStruct((1, H), jnp.float32),
            jax.ShapeDtypeStruct((1, H), jnp.float32),
        ],
        scratch_shapes=[
            pltpu.VMEM((1, H), jnp.float32),
            pltpu.VMEM((1, H), jnp.float32),
        ],
    )(G8, W_hh_T, h0, c0)
    return ys.reshape(n, H), hn, cn


def _edge_pass(y, src, dst, n):
    deg_like = jax.ops.segment_sum(y[src], dst, num_segments=n)
    return deg_like


def kernel(basic_block, edge_index, h0, c0, gcn_W, gcn_b,
           W_ih, W_hh, b_ih, b_hh):
    n = basic_block.shape[0]
    src = edge_index[0]
    dst = edge_index[1]
    deg = jax.ops.segment_sum(
        jnp.ones_like(dst, dtype=jnp.float32), dst, num_segments=n)
    y, dinv = _compute_y(basic_block, gcn_W, deg)
    acc = _edge_pass(y, src, dst, n)
    G = _compute_g(acc, y, dinv, gcn_b, W_ih, b_ih, b_hh)
    ys, hn, cn = _run_lstm(G, W_hh.T, h0, c0)
    return ys, hn, cn


# speedup vs baseline: 17.1485x; 1.8699x over previous
import functools

import jax
import jax.numpy as jnp
from jax import lax
from jax.experimental import pallas as pl
from jax.experimental.pallas import tpu as pltpu
from jax.experimental.pallas import tpu_sc as plsc


H = 128
_NW = 32
_CH = 128


def _sc_mesh():
    return plsc.VectorSubcoreMesh(core_axis_name="c", subcore_axis_name="s")


def _deg_sc(dst3, np_, nch):
    ts = np_ // 16

    @functools.partial(
        pl.kernel,
        out_type=jax.ShapeDtypeStruct((2, np_), jnp.float32),
        mesh=_sc_mesh(),
        scratch_types=[
            pltpu.VMEM((nch, _CH), jnp.int32),
            pltpu.VMEM((_CH,), jnp.float32),
            pltpu.VMEM((ts,), jnp.float32),
            pltpu.VMEM_SHARED((np_,), jnp.float32),
        ],
    )
    def body(dst_hbm, deg_hbm, dst_v, ones_v, st_v, deg_sh):
        cid = lax.axis_index("c")
        sid = lax.axis_index("s")
        wid = cid * 16 + sid

        def fill_ones(i, _):
            ones_v[pl.ds(i * 16, 16)] = jnp.ones((16,), jnp.float32)
            return 0

        lax.fori_loop(0, _CH // 16, fill_ones, 0)

        def fill_zero(i, _):
            st_v[pl.ds(i * 16, 16)] = jnp.zeros((16,), jnp.float32)
            return 0

        lax.fori_loop(0, ts // 16, fill_zero, 0)
        pltpu.sync_copy(st_v, deg_sh.at[pl.ds(sid * ts, ts)])
        plsc.subcore_barrier()

        pltpu.sync_copy(dst_hbm.at[wid], dst_v)

        def chunk(j, _):
            pltpu.sync_copy(ones_v, deg_sh.at[dst_v.at[j]], add=True)
            return 0

        lax.fori_loop(0, nch, chunk, 0)
        plsc.subcore_barrier()
        pltpu.sync_copy(deg_sh.at[pl.ds(sid * ts, ts)], st_v)
        pltpu.sync_copy(st_v, deg_hbm.at[cid, pl.ds(sid * ts, ts)])

    return body(dst3)


def _msg_sc(y, src3, dst3, np_, nch):
    ts = np_ // 16
    sb = ts // 8

    @functools.partial(
        pl.kernel,
        out_type=jax.ShapeDtypeStruct((2, np_, H), jnp.float32),
        mesh=_sc_mesh(),
        scratch_types=[
            pltpu.VMEM((nch, _CH), jnp.int32),
            pltpu.VMEM((nch, _CH), jnp.int32),
            pltpu.VMEM((_CH, H), jnp.float32),
            pltpu.VMEM((sb, H), jnp.float32),
            pltpu.VMEM_SHARED((np_, H), jnp.float32),
            pltpu.SemaphoreType.DMA,
        ],
    )
    def body(y_hbm, src_hbm, dst_hbm, acc_hbm,
             src_v, dst_v, rows_v, zb_v, acc_sh, sem):
        cid = lax.axis_index("c")
        sid = lax.axis_index("s")
        wid = cid * 16 + sid

        def zrow(i, _):
            for l in range(H // 16):
                zb_v[i, pl.ds(l * 16, 16)] = jnp.zeros((16,), jnp.float32)
            return 0

        lax.fori_loop(0, sb, zrow, 0)
        for k in range(8):
            pltpu.sync_copy(zb_v, acc_sh.at[pl.ds(sid * ts + k * sb, sb)])
        plsc.subcore_barrier()

        pltpu.sync_copy(src_hbm.at[wid], src_v)
        pltpu.sync_copy(dst_hbm.at[wid], dst_v)

        def chunk(j, _):
            pltpu.async_copy(y_hbm.at[src_v.at[j]], rows_v, sem).wait()
            pltpu.sync_copy(rows_v, acc_sh.at[dst_v.at[j]], add=True)
            return 0

        lax.fori_loop(0, nch, chunk, 0)
        plsc.subcore_barrier()
        for k in range(8):
            pltpu.sync_copy(acc_sh.at[pl.ds(sid * ts + k * sb, sb)], zb_v)
            pltpu.sync_copy(zb_v, acc_hbm.at[cid, pl.ds(sid * ts + k * sb, sb)])

    return body(y, src3, dst3)


def _y_kernel(x_ref, w_ref, deg_ref, y_ref, dinv_ref):
    deg = deg_ref[0, 0, 0, :] + deg_ref[1, 0, 0, :] + 1.0
    dinv = lax.rsqrt(deg)
    xw = jnp.dot(x_ref[...], w_ref[...], preferred_element_type=jnp.float32)
    y_ref[...] = xw * dinv[:, None]
    dinv_ref[0, 0, :] = dinv


def _compute_y(x, gcn_W, degp):
    n = x.shape[0]
    blk = 1000
    grid = n // blk
    y, dinv = pl.pallas_call(
        _y_kernel,
        grid=(grid,),
        in_specs=[
            pl.BlockSpec((blk, H), lambda i: (i, 0)),
            pl.BlockSpec((H, H), lambda i: (0, 0)),
            pl.BlockSpec((2, 1, 1, blk), lambda i: (0, i, 0, 0)),
        ],
        out_specs=[
            pl.BlockSpec((blk, H), lambda i: (i, 0)),
            pl.BlockSpec((1, 1, blk), lambda i: (i, 0, 0)),
        ],
        out_shape=[
            jax.ShapeDtypeStruct((n, H), jnp.float32),
            jax.ShapeDtypeStruct((grid, 1, blk), jnp.float32),
        ],
    )(x, gcn_W, degp[:, :n].reshape(2, grid, 1, blk))
    return y, dinv


def _g_kernel(acc_ref, y_ref, dinv_ref, gb_ref, wih_ref, b_ref, g_ref):
    dinv = dinv_ref[0, 0, :]
    acc = acc_ref[0] + acc_ref[1]
    x = dinv[:, None] * (acc + y_ref[...]) + gb_ref[...]
    g_ref[...] = (
        lax.dot_general(x, wih_ref[...], (((1,), (1,)), ((), ())),
                        preferred_element_type=jnp.float32)
        + b_ref[...]
    )


def _compute_g(acc, y, dinv, gcn_b, W_ih, b_ih, b_hh):
    n = y.shape[0]
    blk = 1000
    grid = n // blk
    return pl.pallas_call(
        _g_kernel,
        grid=(grid,),
        in_specs=[
            pl.BlockSpec((2, blk, H), lambda i: (0, i, 0)),
            pl.BlockSpec((blk, H), lambda i: (i, 0)),
            pl.BlockSpec((1, 1, blk), lambda i: (i, 0, 0)),
            pl.BlockSpec((1, H), lambda i: (0, 0)),
            pl.BlockSpec((4 * H, H), lambda i: (0, 0)),
            pl.BlockSpec((1, 4 * H), lambda i: (0, 0)),
        ],
        out_specs=pl.BlockSpec((blk, 4 * H), lambda i: (i, 0)),
        out_shape=jax.ShapeDtypeStruct((n, 4 * H), jnp.float32),
    )(acc, y, dinv, gcn_b.reshape(1, H), W_ih,
      (b_ih + b_hh).reshape(1, 4 * H))


def _lstm_kernel(g_ref, whht_ref, h0_ref, c0_ref,
                 ys_ref, hn_ref, cn_ref, h_s, c_s, *, nblk):
    t = pl.program_id(0)

    @pl.when(t == 0)
    def _():
        h_s[...] = h0_ref[...]
        c_s[...] = c0_ref[...]

    whht = whht_ref[...]

    def blk_step(j, carry):
        h, c = carry
        gblk = g_ref[j]
        rows = []
        for k in range(8):
            g = gblk[k:k + 1, :] + jnp.dot(
                h, whht, preferred_element_type=jnp.float32)
            ii = jax.nn.sigmoid(g[:, :H])
            ff = jax.nn.sigmoid(g[:, H:2 * H])
            gg = jnp.tanh(g[:, 2 * H:3 * H])
            oo = jax.nn.sigmoid(g[:, 3 * H:])
            c = ff * c + ii * gg
            h = oo * jnp.tanh(c)
            rows.append(h)
        ys_ref[j] = jnp.concatenate(rows, axis=0)
        return h, c

    h, c = lax.fori_loop(0, nblk, blk_step, (h_s[...], c_s[...]))
    h_s[...] = h
    c_s[...] = c
    hn_ref[...] = h
    cn_ref[...] = c


def _run_lstm(G, W_hh_T, h0, c0):
    n = G.shape[0]
    chunk = 1000
    grid = n // chunk
    nblk = chunk // 8
    G8 = G.reshape(n // 8, 8, 4 * H)
    ys, hn, cn = pl.pallas_call(
        functools.partial(_lstm_kernel, nblk=nblk),
        grid=(grid,),
        in_specs=[
            pl.BlockSpec((nblk, 8, 4 * H), lambda i: (i, 0, 0)),
            pl.BlockSpec((H, 4 * H), lambda i: (0, 0)),
            pl.BlockSpec((1, H), lambda i: (0, 0)),
            pl.BlockSpec((1, H), lambda i: (0, 0)),
        ],
        out_specs=[
            pl.BlockSpec((nblk, 8, H), lambda i: (i, 0, 0)),
            pl.BlockSpec((1, H), lambda i: (0, 0)),
            pl.BlockSpec((1, H), lambda i: (0, 0)),
        ],
        out_shape=[
            jax.ShapeDtypeStruct((n // 8, 8, H), jnp.float32),
            jax.ShapeDtypeStruct((1, H), jnp.float32),
            jax.ShapeDtypeStruct((1, H), jnp.float32),
        ],
        scratch_shapes=[
            pltpu.VMEM((1, H), jnp.float32),
            pltpu.VMEM((1, H), jnp.float32),
        ],
    )(G8, W_hh_T, h0, c0)
    return ys.reshape(n, H), hn, cn


def kernel(basic_block, edge_index, h0, c0, gcn_W, gcn_b,
           W_ih, W_hh, b_ih, b_hh):
    n = basic_block.shape[0]
    e = edge_index.shape[1]
    nch = -(-e // (_NW * _CH))
    ep = _NW * nch * _CH
    np_ = -(-n // 256) * 256
    pad = ep - e
    src3 = jnp.concatenate(
        [edge_index[0], jnp.zeros((pad,), edge_index.dtype)]
    ).reshape(_NW, nch, _CH)
    dst3 = jnp.concatenate(
        [edge_index[1], jnp.full((pad,), n, edge_index.dtype)]
    ).reshape(_NW, nch, _CH)

    degp = _deg_sc(dst3, np_, nch)
    y, dinv = _compute_y(basic_block, gcn_W, degp)
    accp = _msg_sc(y, src3, dst3, np_, nch)
    G = _compute_g(accp[:, :n], y, dinv, gcn_b, W_ih, b_ih, b_hh)
    ys, hn, cn = _run_lstm(G, W_hh.T, h0, c0)
    return ys, hn, cn
